# baseline (speedup 1.0000x reference)
import functools

import jax
import jax.numpy as jnp
from jax import lax
from jax.experimental import pallas as pl
from jax.experimental.pallas import tpu as pltpu
from jax.experimental.pallas import tpu_sc as plsc

D = 64
B = 4096 * 200
NW = 32
BPW = B // NW
C = 128
NCHUNK = BPW // C

_mesh = plsc.VectorSubcoreMesh(core_axis_name="c", subcore_axis_name="s")


@functools.partial(
    pl.kernel,
    mesh=_mesh,
    compiler_params=pltpu.CompilerParams(use_tc_tiling_on_sc=False),
    out_type=jax.ShapeDtypeStruct((B, D), jnp.float32),
    scratch_types=[
        pltpu.VMEM((NCHUNK, C), jnp.int32),
        pltpu.VMEM((NCHUNK, C), jnp.int32),
        pltpu.VMEM((C, D), jnp.float32),
        pltpu.VMEM((C, D), jnp.float32),
        pltpu.VMEM((C, D), jnp.float32),
        pltpu.VMEM((C, D), jnp.float32),
        pltpu.VMEM((C, D), jnp.float32),
        pltpu.VMEM((C, D), jnp.float32),
        pltpu.SemaphoreType.DMA,
        pltpu.SemaphoreType.DMA,
        pltpu.SemaphoreType.DMA,
        pltpu.SemaphoreType.DMA,
        pltpu.SemaphoreType.DMA,
        pltpu.SemaphoreType.DMA,
    ],
)
def _emb_add(x_hbm, c_hbm, w_hbm, out_hbm, ix_v, ic_v,
             rx0, rx1, rc0, rc1, ob0, ob1,
             sgx0, sgx1, sgc0, sgc1, ss0, ss1):
    wid = lax.axis_index("s") * 2 + lax.axis_index("c")
    base = wid * BPW
    rx = (rx0, rx1)
    rc = (rc0, rc1)
    ob = (ob0, ob1)
    sgx = (sgx0, sgx1)
    sgc = (sgc0, sgc1)
    ss = (ss0, ss1)

    pltpu.sync_copy(x_hbm.at[wid], ix_v)
    pltpu.sync_copy(c_hbm.at[wid], ic_v)

    for b in range(2):
        pltpu.async_copy(w_hbm.at[ix_v.at[b]], rx[b], sgx[b])
        pltpu.async_copy(w_hbm.at[ic_v.at[b]], rc[b], sgc[b])

    def pair(k, carry):
        for b in range(2):
            i = 2 * k + b
            pltpu.make_async_copy(w_hbm.at[ix_v.at[i]], rx[b], sgx[b]).wait()
            pltpu.make_async_copy(w_hbm.at[ic_v.at[i]], rc[b], sgc[b]).wait()

            @pl.when(k > 0)
            def _():
                pltpu.make_async_copy(
                    ob[b], out_hbm.at[pl.ds(base, C)], ss[b]).wait()

            def row(r, c2):
                for j in range(D // 16):
                    sl = pl.ds(j * 16, 16)
                    ob[b][r, sl] = rx[b][r, sl] + rc[b][r, sl]
                return c2

            lax.fori_loop(0, C, row, 0, unroll=4)

            @pl.when(i + 2 < NCHUNK)
            def _():
                pltpu.async_copy(w_hbm.at[ix_v.at[i + 2]], rx[b], sgx[b])
                pltpu.async_copy(w_hbm.at[ic_v.at[i + 2]], rc[b], sgc[b])

            pltpu.async_copy(ob[b], out_hbm.at[pl.ds(base + i * C, C)], ss[b])
        return carry

    lax.fori_loop(0, NCHUNK // 2, pair, 0, unroll=False)

    for b in range(2):
        pltpu.make_async_copy(ob[b], out_hbm.at[pl.ds(base, C)], ss[b]).wait()


def kernel(x, constant, weight):
    x32 = x.astype(jnp.int32).reshape(NW, NCHUNK, C)
    c32 = constant.astype(jnp.int32).reshape(NW, NCHUNK, C)
    out = _emb_add(x32, c32, weight)
    return out.reshape(4096, 200, D)
